# baseline (speedup 1.0000x reference)
import jax
import jax.numpy as jnp
from jax.experimental import pallas as pl

B = 4
PB = 2
GRID = B // PB
N = 1024
BN = PB * N
EMB_DIM = 16
HID = 64
GNN = 3
KNN = 5
CLAMP = 2.0
BIG = 1e30

_HI = jax.lax.Precision.HIGHEST


def _silu(x):
    return x * jax.nn.sigmoid(x)


def _dot(a, b, precision=None):
    return jax.lax.dot_general(a, b, (((1,), (0,)), ((), ())),
                               precision=precision,
                               preferred_element_type=jnp.float32)


def _fwd(types_ref, pos_ref, emb_ref,
         ew1_ref, eb1_ref, ew2_ref, eb2_ref, gw_ref, gb_ref,
         cw1_ref, cb1_ref, cw2_ref, cb2_ref, cs_ref,
         nw1_ref, nb1_ref, nw2_ref, nb2_ref,
         mw1_ref, mb1_ref, mw2_ref, mb2_ref, mw3_ref, mb3_ref,
         o_ref):
    f32 = jnp.float32
    tcol = types_ref[0]
    oh6 = (tcol == jax.lax.broadcasted_iota(jnp.int32, (1, 6), 1).astype(f32)
           ).astype(f32)
    feats = _dot(oh6, emb_ref[...], _HI)
    C = pos_ref[0]

    eye3 = (jax.lax.broadcasted_iota(jnp.int32, (3, 3), 0)
            == jax.lax.broadcasted_iota(jnp.int32, (3, 3), 1)).astype(f32)
    iota_nn = jax.lax.broadcasted_iota(jnp.int32, (N, N), 1).astype(f32)
    row_iota = jax.lax.broadcasted_iota(jnp.int32, (N, N), 0).astype(f32)
    diag = iota_nn == row_iota

    for l in range(GNN):
        dblocks = []
        for b in range(PB):
            Cb = C[b * N:(b + 1) * N]
            CTb = jax.lax.dot_general(eye3, Cb, (((1,), (1,)), ((), ())),
                                      precision=_HI,
                                      preferred_element_type=f32)
            dx = Cb[:, 0:1] - CTb[0:1, :]
            dy = Cb[:, 1:2] - CTb[1:2, :]
            dz = Cb[:, 2:3] - CTb[2:3, :]
            db = dx * dx + dy * dy + dz * dz
            dblocks.append(jnp.where(diag, f32(BIG), db))
        dwork = jnp.concatenate(dblocks, axis=0)

        feats_ext = jnp.concatenate([feats, C], axis=1)
        dists = [jnp.zeros((BN, 1), f32)]
        fjs = [feats]
        pjs = [C]
        for _ in range(KNN - 1):
            m = jnp.min(dwork, axis=1, keepdims=True)
            ohb = dwork == m
            ohf = ohb.astype(f32)
            g = jnp.concatenate(
                [_dot(ohf[b * N:(b + 1) * N],
                      feats_ext[b * N:(b + 1) * N]) for b in range(PB)],
                axis=0)
            dists.append(m)
            fjs.append(g[:, :EMB_DIM])
            pjs.append(g[:, EMB_DIM:EMB_DIM + 3])
            dwork = jnp.where(ohb, f32(BIG), dwork)

        E = jnp.concatenate(
            [jnp.concatenate([feats, fjs[k], dists[k]], axis=1)
             for k in range(KNN)], axis=0)
        h1 = _silu(_dot(E, ew1_ref[l]) + eb1_ref[l])
        mij = _silu(_dot(h1, ew2_ref[l]) + eb2_ref[l])
        gate = jax.nn.sigmoid(_dot(mij, gw_ref[l]) + gb_ref[l])
        mij = mij * gate
        hc = _silu(_dot(mij, cw1_ref[l]) + cb1_ref[l])
        w = _dot(hc, cw2_ref[l]) + cb2_ref[l]
        w = jnp.clip(w, -CLAMP, CLAMP)

        cs = cs_ref[l]
        delta = jnp.zeros((BN, 3), f32)
        for k in range(KNN):
            rel = C - pjs[k]
            nrm = jnp.sqrt(jnp.sum(rel * rel, axis=1, keepdims=True))
            reln = rel / jnp.maximum(nrm, 1e-8) * cs
            delta = delta + w[k * BN:(k + 1) * BN] * reln

        m_i = mij[0:BN]
        for k in range(1, KNN):
            m_i = m_i + mij[k * BN:(k + 1) * BN]

        node_in = jnp.concatenate([feats, m_i], axis=1)
        n1 = _silu(_dot(node_in, nw1_ref[l]) + nb1_ref[l])
        feats = _dot(n1, nw2_ref[l]) + nb2_ref[l] + feats
        C = delta + C

    pooled = jnp.concatenate(
        [jnp.sum(feats[b * N:(b + 1) * N], axis=0, keepdims=True)
         for b in range(PB)], axis=0) / f32(N)
    h = jax.nn.relu(_dot(pooled, mw1_ref[...]) + mb1_ref[...])
    h = jax.nn.relu(_dot(h, mw2_ref[...]) + mb2_ref[...])
    o_ref[0] = _dot(h, mw3_ref[...]) + mb3_ref[...]


def kernel(atom_types, pos, mask, emb, layers, mlp):
    del mask
    f32 = jnp.float32
    types_f = atom_types.astype(f32).reshape(GRID, BN, 1)
    pos_f = pos.reshape(GRID, BN, 3)

    def stk(name):
        return jnp.stack([p[name] for p in layers])

    def stkb(name):
        return jnp.stack([p[name].reshape(1, -1) for p in layers])

    ops = [
        types_f, pos_f, emb,
        stk("ew1"), stkb("eb1"), stk("ew2"), stkb("eb2"),
        stk("gw"), stkb("gb"),
        stk("cw1"), stkb("cb1"), stk("cw2"), stkb("cb2"), stkb("cscale"),
        stk("nw1"), stkb("nb1"), stk("nw2"), stkb("nb2"),
        mlp["w1"], mlp["b1"].reshape(1, -1),
        mlp["w2"], mlp["b2"].reshape(1, -1),
        mlp["w3"], mlp["b3"].reshape(1, -1),
    ]

    def full_spec(a):
        nd = a.ndim
        return pl.BlockSpec(a.shape, lambda g, _nd=nd: (0,) * _nd)

    in_specs = [
        pl.BlockSpec((1, BN, 1), lambda g: (g, 0, 0)),
        pl.BlockSpec((1, BN, 3), lambda g: (g, 0, 0)),
    ] + [full_spec(a) for a in ops[2:]]

    out = pl.pallas_call(
        _fwd,
        grid=(GRID,),
        in_specs=in_specs,
        out_specs=pl.BlockSpec((1, PB, 1), lambda g: (g, 0, 0)),
        out_shape=jax.ShapeDtypeStruct((GRID, PB, 1), f32),
    )(*ops)
    return out.reshape(B, 1)


# speedup vs baseline: 1.2998x; 1.2998x over previous
import functools

import jax
import jax.numpy as jnp
from jax.experimental import pallas as pl

N = 1024
EMB_DIM = 16
HID = 64
GNN = 3
KNN = 5
CLAMP = 2.0
BIG = 1e30

_HI = jax.lax.Precision.HIGHEST


def _silu(x):
    return x * jax.nn.sigmoid(x)


def _dot(a, b, precision=None):
    return jax.lax.dot_general(a, b, (((1,), (0,)), ((), ())),
                               precision=precision,
                               preferred_element_type=jnp.float32)


def _fwd(types_ref, pos_ref, emb_ref,
         ew1_ref, eb1_ref, ew2_ref, eb2_ref, gw_ref, gb_ref,
         cw1_ref, cb1_ref, cw2_ref, cb2_ref, cs_ref,
         nw1_ref, nb1_ref, nw2_ref, nb2_ref,
         mw1_ref, mb1_ref, mw2_ref, mb2_ref, mw3_ref, mb3_ref,
         o_ref):
    f32 = jnp.float32
    tcol = types_ref[0]
    oh6 = (tcol == jax.lax.broadcasted_iota(jnp.int32, (1, 6), 1).astype(f32)
           ).astype(f32)
    feats = _dot(oh6, emb_ref[...])
    C = pos_ref[0]

    eye3 = (jax.lax.broadcasted_iota(jnp.int32, (3, 3), 0)
            == jax.lax.broadcasted_iota(jnp.int32, (3, 3), 1)).astype(f32)
    iota_nn = jax.lax.broadcasted_iota(jnp.int32, (N, N), 1).astype(f32)
    row_iota = jax.lax.broadcasted_iota(jnp.int32, (N, N), 0).astype(f32)
    diag_big = jnp.where(iota_nn == row_iota, f32(BIG), f32(0.0))

    for l in range(GNN):
        CT = jax.lax.dot_general(eye3, C, (((1,), (1,)), ((), ())),
                                 precision=_HI, preferred_element_type=f32)
        dx = C[:, 0:1] - CT[0:1, :]
        dy = C[:, 1:2] - CT[1:2, :]
        dz = C[:, 2:3] - CT[2:3, :]
        dwork = dx * dx + dy * dy + dz * dz + diag_big

        feats_ext = jnp.concatenate([feats, C], axis=1)
        dists = [jnp.zeros((N, 1), f32)]
        fjs = [feats]
        pjs = [C]
        for _ in range(KNN - 1):
            m = jnp.min(dwork, axis=1, keepdims=True)
            ohb = dwork == m
            g = _dot(ohb.astype(f32), feats_ext)
            dists.append(m)
            fjs.append(g[:, :EMB_DIM])
            pjs.append(g[:, EMB_DIM:EMB_DIM + 3])
            dwork = jnp.where(ohb, f32(BIG), dwork)

        E = jnp.concatenate(
            [jnp.concatenate([feats, fjs[k], dists[k]], axis=1)
             for k in range(KNN)], axis=0)
        h1 = _silu(_dot(E, ew1_ref[l]) + eb1_ref[l])
        mij = _silu(_dot(h1, ew2_ref[l]) + eb2_ref[l])
        gate = jax.nn.sigmoid(_dot(mij, gw_ref[l]) + gb_ref[l])
        mij = mij * gate
        hc = _silu(_dot(mij, cw1_ref[l]) + cb1_ref[l])
        w = _dot(hc, cw2_ref[l]) + cb2_ref[l]
        w = jnp.clip(w, -CLAMP, CLAMP)

        cs = cs_ref[l]
        delta = jnp.zeros((N, 3), f32)
        for k in range(KNN):
            rel = C - pjs[k]
            nrm = jnp.sqrt(dists[k])
            reln = rel / jnp.maximum(nrm, 1e-8) * cs
            delta = delta + w[k * N:(k + 1) * N] * reln

        m_i = mij[0:N]
        for k in range(1, KNN):
            m_i = m_i + mij[k * N:(k + 1) * N]

        node_in = jnp.concatenate([feats, m_i], axis=1)
        n1 = _silu(_dot(node_in, nw1_ref[l]) + nb1_ref[l])
        feats = _dot(n1, nw2_ref[l]) + nb2_ref[l] + feats
        C = delta + C

    pooled = jnp.sum(feats, axis=0, keepdims=True) / f32(N)
    h = jax.nn.relu(_dot(pooled, mw1_ref[...]) + mb1_ref[...])
    h = jax.nn.relu(_dot(h, mw2_ref[...]) + mb2_ref[...])
    o_ref[0] = _dot(h, mw3_ref[...]) + mb3_ref[...]


def kernel(atom_types, pos, mask, emb, layers, mlp):
    B, n = atom_types.shape
    del mask
    f32 = jnp.float32
    types_f = atom_types.astype(f32).reshape(B, n, 1)

    def stk(name):
        return jnp.stack([p[name] for p in layers])

    def stkb(name):
        return jnp.stack([p[name].reshape(1, -1) for p in layers])

    ops = [
        types_f, pos, emb,
        stk("ew1"), stkb("eb1"), stk("ew2"), stkb("eb2"),
        stk("gw"), stkb("gb"),
        stk("cw1"), stkb("cb1"), stk("cw2"), stkb("cb2"), stkb("cscale"),
        stk("nw1"), stkb("nb1"), stk("nw2"), stkb("nb2"),
        mlp["w1"], mlp["b1"].reshape(1, -1),
        mlp["w2"], mlp["b2"].reshape(1, -1),
        mlp["w3"], mlp["b3"].reshape(1, -1),
    ]

    def full_spec(a):
        nd = a.ndim
        return pl.BlockSpec(a.shape, lambda b, _nd=nd: (0,) * _nd)

    in_specs = [
        pl.BlockSpec((1, n, 1), lambda b: (b, 0, 0)),
        pl.BlockSpec((1, n, 3), lambda b: (b, 0, 0)),
    ] + [full_spec(a) for a in ops[2:]]

    out = pl.pallas_call(
        _fwd,
        grid=(B,),
        in_specs=in_specs,
        out_specs=pl.BlockSpec((1, 1, 1), lambda b: (b, 0, 0)),
        out_shape=jax.ShapeDtypeStruct((B, 1, 1), f32),
    )(*ops)
    return out.reshape(B, 1)
